# baseline (speedup 1.0000x reference)
import jax
import jax.numpy as jnp
from jax.experimental import pallas as pl
from jax.experimental.pallas import tpu as pltpu

_N, _D = 2048, 128
_TR = 256
_K = 256
_NQ = _N // 16
_MARGIN = 1.0
_EPS = 1e-6
_BIG = 1e6


def _loss_body(orig_ref, aug_ref, l_ref, *rest):
    adj_refs = rest[:16]
    out_ref = rest[16]
    af_ref, bp_ref, bn_ref = rest[17], rest[18], rest[19]
    i = pl.program_id(0)
    dn = (((1,), (1,)), ((), ()))

    @pl.when(i == 0)
    def _():
        o = orig_ref[...]
        g = aug_ref[...]
        lv = l_ref[...]
        no = jnp.sum(o * o, axis=1, keepdims=True)
        so = jnp.sum(o, axis=1, keepdims=True)
        na = jnp.sum(g * g, axis=1, keepdims=True)
        sa = jnp.sum(g, axis=1, keepdims=True)
        rp = no - (2.0 * _EPS) * so
        rn = (na - (2.0 * _EPS) * sa
              + _BIG * (lv != 1).astype(jnp.float32))
        base = no + (2.0 * _EPS) * so + _D * _EPS * _EPS
        big_m0 = _BIG * (lv != 0).astype(jnp.float32)
        ones_col = jnp.ones((_N, 1), jnp.bfloat16)

        af_ref[...] = jnp.zeros((_N, _K), jnp.bfloat16)
        af_ref[:, 0:_D] = (o * -2.0).astype(jnp.bfloat16)
        af_ref[:, _D:_D + 1] = base.astype(jnp.bfloat16)
        af_ref[:, _D + 1:_D + 2] = ones_col
        af_ref[:, _D + 2:_D + 3] = big_m0.astype(jnp.bfloat16)

        bp_ref[...] = jnp.zeros((_N, _K), jnp.bfloat16)
        bp_ref[:, 0:_D] = o.astype(jnp.bfloat16)
        bp_ref[:, _D:_D + 1] = ones_col
        bp_ref[:, _D + 1:_D + 2] = rp.astype(jnp.bfloat16)

        bn_ref[...] = jnp.zeros((_N, _K), jnp.bfloat16)
        bn_ref[:, 0:_D] = g.astype(jnp.bfloat16)
        bn_ref[:, _D:_D + 1] = ones_col
        bn_ref[:, _D + 1:_D + 2] = rn.astype(jnp.bfloat16)
        bn_ref[:, _D + 2:_D + 3] = ones_col
        out_ref[0] = 0.0

    av = af_ref[pl.ds(i * _TR, _TR), :]
    pos_sq = jax.lax.dot_general(av, bp_ref[...], dn,
                                 preferred_element_type=jnp.float32)
    neg_sq = jax.lax.dot_general(av, bn_ref[...], dn,
                                 preferred_element_type=jnp.float32)

    mp = jnp.maximum(pos_sq, 1e-12)
    mn = jnp.maximum(neg_sq, 1e-12)
    d_pos = mp * jax.lax.rsqrt(mp)
    d_neg = mn * jax.lax.rsqrt(mn)
    hinge = jnp.maximum(d_pos - d_neg + _MARGIN, 0.0)
    acc = jnp.float32(0.0)
    for q in range(16):
        acc += jnp.sum(adj_refs[q][...] * hinge[:, q * _NQ:(q + 1) * _NQ])
    out_ref[0] += acc


def kernel(orig, aug, l, adj):
    lc = l.reshape(_N, 1)
    out = pl.pallas_call(
        _loss_body,
        grid=(_N // _TR,),
        in_specs=[
            pl.BlockSpec((_N, _D), lambda i: (0, 0)),
            pl.BlockSpec((_N, _D), lambda i: (0, 0)),
            pl.BlockSpec((_N, 1), lambda i: (0, 0)),
            *[pl.BlockSpec((_TR, _NQ), (lambda q: (lambda i: (i, q)))(q))
              for q in range(16)],
        ],
        out_specs=pl.BlockSpec(memory_space=pltpu.SMEM),
        out_shape=jax.ShapeDtypeStruct((1,), jnp.float32),
        scratch_shapes=[
            pltpu.VMEM((_N, _K), jnp.bfloat16),
            pltpu.VMEM((_N, _K), jnp.bfloat16),
            pltpu.VMEM((_N, _K), jnp.bfloat16),
        ],
        compiler_params=pltpu.CompilerParams(
            dimension_semantics=("arbitrary",)),
    )(orig, aug, lc, *([adj] * 16))
    return out[0]


# speedup vs baseline: 1.0862x; 1.0862x over previous
import jax
import jax.numpy as jnp
from jax.experimental import pallas as pl
from jax.experimental.pallas import tpu as pltpu

_N, _D = 2048, 128
_TR = 512
_K = 256
_NQ = _N // 16
_MARGIN = 1.0
_EPS = 1e-6
_BIG = 1e6


def _loss_body(orig_ref, aug_ref, l_ref, *rest):
    adj_refs = rest[:16]
    out_ref = rest[16]
    af_ref, bp_ref, bn_ref = rest[17], rest[18], rest[19]
    i = pl.program_id(0)
    dn = (((1,), (1,)), ((), ()))

    @pl.when(i == 0)
    def _():
        o = orig_ref[...]
        g = aug_ref[...]
        lv = l_ref[...]
        no = jnp.sum(o * o, axis=1, keepdims=True)
        so = jnp.sum(o, axis=1, keepdims=True)
        na = jnp.sum(g * g, axis=1, keepdims=True)
        sa = jnp.sum(g, axis=1, keepdims=True)
        rp = no - (2.0 * _EPS) * so
        rn = (na - (2.0 * _EPS) * sa
              + _BIG * (lv != 1).astype(jnp.float32))
        base = no + (2.0 * _EPS) * so + _D * _EPS * _EPS
        big_m0 = _BIG * (lv != 0).astype(jnp.float32)
        ones_col = jnp.ones((_N, 1), jnp.bfloat16)

        af_ref[...] = jnp.zeros((_N, _K), jnp.bfloat16)
        af_ref[:, 0:_D] = (o * -2.0).astype(jnp.bfloat16)
        af_ref[:, _D:_D + 1] = base.astype(jnp.bfloat16)
        af_ref[:, _D + 1:_D + 2] = ones_col
        af_ref[:, _D + 2:_D + 3] = big_m0.astype(jnp.bfloat16)

        bp_ref[...] = jnp.zeros((_N, _K), jnp.bfloat16)
        bp_ref[:, 0:_D] = o.astype(jnp.bfloat16)
        bp_ref[:, _D:_D + 1] = ones_col
        bp_ref[:, _D + 1:_D + 2] = rp.astype(jnp.bfloat16)

        bn_ref[...] = jnp.zeros((_N, _K), jnp.bfloat16)
        bn_ref[:, 0:_D] = g.astype(jnp.bfloat16)
        bn_ref[:, _D:_D + 1] = ones_col
        bn_ref[:, _D + 1:_D + 2] = rn.astype(jnp.bfloat16)
        bn_ref[:, _D + 2:_D + 3] = ones_col
        out_ref[0] = 0.0

    av = af_ref[pl.ds(i * _TR, _TR), :]
    pos_sq = jax.lax.dot_general(av, bp_ref[...], dn,
                                 preferred_element_type=jnp.float32)
    neg_sq = jax.lax.dot_general(av, bn_ref[...], dn,
                                 preferred_element_type=jnp.float32)

    mp = jnp.maximum(pos_sq, 1e-12)
    mn = jnp.maximum(neg_sq, 1e-12)
    d_pos = mp * jax.lax.rsqrt(mp)
    d_neg = mn * jax.lax.rsqrt(mn)
    hinge = jnp.maximum(d_pos - d_neg + _MARGIN, 0.0)
    acc = jnp.float32(0.0)
    for q in range(16):
        acc += jnp.sum(adj_refs[q][...] * hinge[:, q * _NQ:(q + 1) * _NQ])
    out_ref[0] += acc


def kernel(orig, aug, l, adj):
    lc = l.reshape(_N, 1)
    out = pl.pallas_call(
        _loss_body,
        grid=(_N // _TR,),
        in_specs=[
            pl.BlockSpec((_N, _D), lambda i: (0, 0)),
            pl.BlockSpec((_N, _D), lambda i: (0, 0)),
            pl.BlockSpec((_N, 1), lambda i: (0, 0)),
            *[pl.BlockSpec((_TR, _NQ), (lambda q: (lambda i: (i, q)))(q))
              for q in range(16)],
        ],
        out_specs=pl.BlockSpec(memory_space=pltpu.SMEM),
        out_shape=jax.ShapeDtypeStruct((1,), jnp.float32),
        scratch_shapes=[
            pltpu.VMEM((_N, _K), jnp.bfloat16),
            pltpu.VMEM((_N, _K), jnp.bfloat16),
            pltpu.VMEM((_N, _K), jnp.bfloat16),
        ],
        compiler_params=pltpu.CompilerParams(
            dimension_semantics=("arbitrary",)),
    )(orig, aug, lc, *([adj] * 16))
    return out[0]
